# Initial kernel scaffold; baseline (speedup 1.0000x reference)
import jax
import jax.numpy as jnp
from jax.experimental import pallas as pl


def kernel(input_tensor, target):
    raise NotImplementedError("write your pallas kernel here")


# speedup vs baseline: 1.8617x; 1.8617x over previous
import functools

import jax
import jax.numpy as jnp
from jax import lax
from jax.experimental import pallas as pl
from jax.experimental.pallas import tpu as pltpu
from jax.experimental.pallas import tpu_sc as plsc

N, C, H, W = 8, 192, 56, 56
NP = H * W
NV = NP // 16
NCH = N * C
NWORK = 32
CPW = NCH // NWORK
MININT = jnp.int32(-(2 ** 31))
MASK5 = jnp.int32(31)

_mesh = plsc.VectorSubcoreMesh(core_axis_name="c", subcore_axis_name="s")


@functools.partial(
    pl.kernel,
    out_type=jax.ShapeDtypeStruct((NWORK, 16), jnp.float32),
    mesh=_mesh,
    scratch_types=[
        pltpu.VMEM((NP,), jnp.float32),
        pltpu.VMEM((NP,), jnp.int32),
        pltpu.VMEM((NP,), jnp.int32),
        pltpu.VMEM((NP,), jnp.int32),
        pltpu.VMEM((32,), jnp.int32),
        pltpu.VMEM((32,), jnp.int32),
        pltpu.VMEM((16,), jnp.float32),
    ],
    compiler_params=pltpu.CompilerParams(needs_layout_passes=False),
)
def _hist_loss_kernel(s_hbm, r_hbm, out_hbm, rowf, bufA, bufB, sortS, hist, fill, accv):
    cid = lax.axis_index("c")
    sid = lax.axis_index("s")
    wid = sid * 2 + cid
    zero16 = jnp.zeros((16,), jnp.int32)

    def to_sortable(b):
        return lax.bitwise_xor(b, lax.bitwise_or(lax.shift_right_arithmetic(b, 31), MININT))

    def from_sortable(k):
        b = lax.bitwise_xor(
            k, lax.bitwise_or(lax.bitwise_not(lax.shift_right_arithmetic(k, 31)), MININT)
        )
        return plsc.bitcast(b, jnp.float32)

    def load_convert():
        def body(i, c):
            x = rowf[pl.ds(i * 16, 16)]
            bufA[pl.ds(i * 16, 16)] = to_sortable(plsc.bitcast(x, jnp.int32))
            return c

        lax.fori_loop(0, NV, body, 0)

    def radix_pass(src, dst, sh):
        shift = jnp.int32(sh)
        hist[pl.ds(0, 16)] = zero16
        hist[pl.ds(16, 16)] = zero16

        def hbody(i, c):
            k = src[pl.ds(i * 16, 16)]
            d = lax.bitwise_and(lax.shift_right_logical(k, shift), MASK5)
            cnt, last = plsc.scan_count(d)
            cur = plsc.load_gather(hist, [d])
            plsc.store_scatter(hist, [d], cur + cnt, mask=last)
            return c

        lax.fori_loop(0, NV, hbody, 0)

        h0 = hist[pl.ds(0, 16)]
        h1 = hist[pl.ds(16, 16)]
        e0 = plsc.cumsum(h0) - h0
        e1 = plsc.cumsum(h1) - h1 + jnp.sum(h0)
        fill[pl.ds(0, 16)] = e0
        fill[pl.ds(16, 16)] = e1

        def pbody(i, c):
            k = src[pl.ds(i * 16, 16)]
            d = lax.bitwise_and(lax.shift_right_logical(k, shift), MASK5)
            cnt, last = plsc.scan_count(d)
            cur = plsc.load_gather(fill, [d])
            plsc.store_scatter(dst, [cur + cnt - 1], k)
            plsc.store_scatter(fill, [d], cur + cnt, mask=last)
            return c

        lax.fori_loop(0, NV, pbody, 0)

    def sort_row(dst_final):
        load_convert()
        for p in range(7):
            src = bufA if p % 2 == 0 else bufB
            dst = dst_final if p == 6 else (bufB if p % 2 == 0 else bufA)
            radix_pass(src, dst, 5 * p)

    def chan_body(t, acc):
        ch = wid * CPW + t
        pltpu.sync_copy(s_hbm.at[ch], rowf)
        sort_row(sortS)
        pltpu.sync_copy(r_hbm.at[ch], rowf)
        sort_row(bufB)

        def dbody(i, a):
            fa = from_sortable(sortS[pl.ds(i * 16, 16)])
            fb = from_sortable(bufB[pl.ds(i * 16, 16)])
            d = fa - fb
            return a + d * d

        return lax.fori_loop(0, NV, dbody, acc)

    acc = lax.fori_loop(0, CPW, chan_body, jnp.zeros((16,), jnp.float32))
    accv[...] = acc
    pltpu.sync_copy(accv, out_hbm.at[wid])


def kernel(input_tensor, target):
    s = input_tensor.reshape(NCH, NP)
    r = target.reshape(NCH, NP)
    partials = _hist_loss_kernel(s, r)
    loss = jnp.sum(partials) / jnp.float32(N * C * H * W)
    return input_tensor, loss


# speedup vs baseline: 4.3200x; 2.3204x over previous
import functools

import jax
import jax.numpy as jnp
from jax import lax
from jax.experimental import pallas as pl
from jax.experimental.pallas import tpu as pltpu
from jax.experimental.pallas import tpu_sc as plsc

N, C, H, W = 8, 192, 56, 56
NP = H * W
NV = NP // 16
NCH = N * C
NWORK = 32
CPW = NCH // NWORK
NB = 256
NCHUNK = 4
VPC = NV // NCHUNK
CE = VPC * 16
MININT = jnp.int32(-(2 ** 31))
MASK8 = jnp.int32(255)

_mesh = plsc.VectorSubcoreMesh(core_axis_name="c", subcore_axis_name="s")


@functools.partial(
    pl.kernel,
    out_type=jax.ShapeDtypeStruct((NWORK, 16), jnp.float32),
    mesh=_mesh,
    scratch_types=[
        pltpu.VMEM((NP,), jnp.float32),
        pltpu.VMEM((NP,), jnp.int32),
        pltpu.VMEM((NP,), jnp.int32),
        pltpu.VMEM((NP,), jnp.int32),
        pltpu.VMEM((NCHUNK * NB,), jnp.int32),
        pltpu.VMEM((NCHUNK * NB,), jnp.int32),
        pltpu.VMEM((NB,), jnp.int32),
        pltpu.VMEM((NB,), jnp.int32),
        pltpu.VMEM((NB,), jnp.int32),
        pltpu.VMEM((NB,), jnp.int32),
        pltpu.VMEM((16,), jnp.int32),
        pltpu.VMEM((16,), jnp.float32),
    ],
    compiler_params=pltpu.CompilerParams(needs_layout_passes=False),
)
def _hist_loss_kernel(
    s_hbm, r_hbm, out_hbm,
    rowf, bufA, bufB, sortS, HA, HB, fill0, fill1, fill2, fill3, tmp, accv,
):
    cid = lax.axis_index("c")
    sid = lax.axis_index("s")
    wid = sid * 2 + cid
    zero16 = jnp.zeros((16,), jnp.int32)
    ones16 = jnp.ones((16,), jnp.int32)
    idx15 = jnp.full((16,), 15, jnp.int32)
    fills = (fill0, fill1, fill2, fill3)
    hists = (HA, HB)

    def to_sortable(b):
        return lax.bitwise_xor(b, lax.bitwise_or(lax.shift_right_arithmetic(b, 31), MININT))

    def from_sortable(k):
        b = lax.bitwise_xor(
            k, lax.bitwise_or(lax.bitwise_not(lax.shift_right_arithmetic(k, 31)), MININT)
        )
        return plsc.bitcast(b, jnp.float32)

    def zero_hists():
        def body(j, c):
            HA[pl.ds(j * 16, 16)] = zero16
            HB[pl.ds(j * 16, 16)] = zero16
            return c

        lax.fori_loop(0, NCHUNK * NB // 16, body, 0)

    def conv_hist(hist):
        def body(i, c):
            for u in range(NCHUNK):
                off = (u * VPC + i) * 16
                k = to_sortable(plsc.bitcast(rowf[pl.ds(off, 16)], jnp.int32))
                bufA[pl.ds(off, 16)] = k
                d = lax.bitwise_and(k, MASK8)
                plsc.addupdate_scatter(hist, [d + jnp.int32(u * NB)], ones16)
            return c

        lax.fori_loop(0, VPC, body, 0)

    def scan_fills(hist):
        def body(j, base):
            h0 = hist[pl.ds(j * 16, 16)]
            h1 = hist[pl.ds(NB + j * 16, 16)]
            h2 = hist[pl.ds(2 * NB + j * 16, 16)]
            h3 = hist[pl.ds(3 * NB + j * 16, 16)]
            hist[pl.ds(j * 16, 16)] = zero16
            hist[pl.ds(NB + j * 16, 16)] = zero16
            hist[pl.ds(2 * NB + j * 16, 16)] = zero16
            hist[pl.ds(3 * NB + j * 16, 16)] = zero16
            g = (h0 + h1) + (h2 + h3)
            e = plsc.cumsum(g)
            b0 = base + (e - g)
            b1 = b0 + h0
            b2 = b1 + h1
            b3 = b2 + h2
            fill0[pl.ds(j * 16, 16)] = b0
            fill1[pl.ds(j * 16, 16)] = b1
            fill2[pl.ds(j * 16, 16)] = b2
            fill3[pl.ds(j * 16, 16)] = b3
            tmp[...] = base + e
            return plsc.load_gather(tmp, [idx15])

        lax.fori_loop(0, NB // 16, body, zero16)

    def radix_pass(src, dst, p, hist_next):
        shift = jnp.int32(8 * p)
        shift2 = jnp.int32(8 * (p + 1))

        def body(i, c):
            for u in range(NCHUNK):
                off = (u * VPC + i) * 16
                k = src[pl.ds(off, 16)]
                d = lax.bitwise_and(lax.shift_right_logical(k, shift), MASK8)
                cnt, last = plsc.scan_count(d)
                cur = plsc.load_gather(fills[u], [d])
                nxt = cur + cnt
                pos = nxt - 1
                plsc.store_scatter(dst, [pos], k)
                plsc.store_scatter(fills[u], [d], nxt, mask=last)
                if hist_next is not None:
                    d2 = lax.bitwise_and(lax.shift_right_logical(k, shift2), MASK8)
                    chunk = (
                        (pos >= CE).astype(jnp.int32)
                        + (pos >= 2 * CE).astype(jnp.int32)
                        + (pos >= 3 * CE).astype(jnp.int32)
                    )
                    plsc.addupdate_scatter(
                        hist_next, [lax.bitwise_or(lax.shift_left(chunk, 8), d2)], ones16
                    )
            return c

        lax.fori_loop(0, VPC, body, 0)

    def sort_row(dst_final):
        conv_hist(HA)
        srcs = (bufA, bufB, bufA, bufB)
        dsts = (bufB, bufA, bufB, dst_final)
        for p in range(4):
            scan_fills(hists[p % 2])
            radix_pass(srcs[p], dsts[p], p, hists[1 - p % 2] if p < 3 else None)

    def chan_body(t, acc):
        ch = wid * CPW + t
        pltpu.sync_copy(s_hbm.at[ch], rowf)
        sort_row(sortS)
        pltpu.sync_copy(r_hbm.at[ch], rowf)
        sort_row(bufA)

        def dbody(i, a):
            fa = from_sortable(sortS[pl.ds(i * 16, 16)])
            fb = from_sortable(bufA[pl.ds(i * 16, 16)])
            d = fa - fb
            return a + d * d

        return lax.fori_loop(0, NV, dbody, acc)

    zero_hists()
    acc = lax.fori_loop(0, CPW, chan_body, jnp.zeros((16,), jnp.float32))
    accv[...] = acc
    pltpu.sync_copy(accv, out_hbm.at[wid])


def kernel(input_tensor, target):
    s = input_tensor.reshape(NCH, NP)
    r = target.reshape(NCH, NP)
    partials = _hist_loss_kernel(s, r)
    loss = jnp.sum(partials) / jnp.float32(N * C * H * W)
    return input_tensor, loss


# speedup vs baseline: 4.3228x; 1.0007x over previous
import functools

import jax
import jax.numpy as jnp
from jax import lax
from jax.experimental import pallas as pl
from jax.experimental.pallas import tpu as pltpu
from jax.experimental.pallas import tpu_sc as plsc

N, C, H, W = 8, 192, 56, 56
NP = H * W
NV = NP // 16
NCH = N * C
NWORK = 32
CPW = NCH // NWORK
NB = 256
NCHUNK = 7
VPC = NV // NCHUNK
CE = VPC * 16
CMUL = jnp.int32((1 << 26) // CE + 1)
MININT = jnp.int32(-(2 ** 31))
MASK8 = jnp.int32(255)

_mesh = plsc.VectorSubcoreMesh(core_axis_name="c", subcore_axis_name="s")


@functools.partial(
    pl.kernel,
    out_type=jax.ShapeDtypeStruct((NWORK, 16), jnp.float32),
    mesh=_mesh,
    scratch_types=[
        pltpu.VMEM((NP,), jnp.float32),
        pltpu.VMEM((NP,), jnp.int32),
        pltpu.VMEM((NP,), jnp.int32),
        pltpu.VMEM((NP,), jnp.int32),
        pltpu.VMEM((NCHUNK * NB,), jnp.int32),
        pltpu.VMEM((NCHUNK * NB,), jnp.int32),
    ]
    + [pltpu.VMEM((NB,), jnp.int32) for _ in range(NCHUNK)]
    + [
        pltpu.VMEM((16,), jnp.int32),
        pltpu.VMEM((16,), jnp.float32),
    ],
    compiler_params=pltpu.CompilerParams(needs_layout_passes=False),
)
def _hist_loss_kernel(
    s_hbm, r_hbm, out_hbm,
    rowf, bufA, bufB, sortS, HA, HB, *rest,
):
    fills = rest[:NCHUNK]
    tmp, accv = rest[NCHUNK], rest[NCHUNK + 1]
    cid = lax.axis_index("c")
    sid = lax.axis_index("s")
    wid = sid * 2 + cid
    zero16 = jnp.zeros((16,), jnp.int32)
    ones16 = jnp.ones((16,), jnp.int32)
    idx15 = jnp.full((16,), 15, jnp.int32)
    hists = (HA, HB)

    def to_sortable(b):
        return lax.bitwise_xor(b, lax.bitwise_or(lax.shift_right_arithmetic(b, 31), MININT))

    def from_sortable(k):
        b = lax.bitwise_xor(
            k, lax.bitwise_or(lax.bitwise_not(lax.shift_right_arithmetic(k, 31)), MININT)
        )
        return plsc.bitcast(b, jnp.float32)

    def zero_hists():
        def body(j, c):
            HA[pl.ds(j * 16, 16)] = zero16
            HB[pl.ds(j * 16, 16)] = zero16
            return c

        lax.fori_loop(0, NCHUNK * NB // 16, body, 0)

    def conv_hist(hist):
        def body(i, c):
            for u in range(NCHUNK):
                off = (u * VPC + i) * 16
                k = to_sortable(plsc.bitcast(rowf[pl.ds(off, 16)], jnp.int32))
                bufA[pl.ds(off, 16)] = k
                d = lax.bitwise_and(k, MASK8)
                plsc.addupdate_scatter(hist, [d + jnp.int32(u * NB)], ones16)
            return c

        lax.fori_loop(0, VPC, body, 0, unroll=2)

    def scan_fills(hist):
        def body(j, base):
            h = []
            for u in range(NCHUNK):
                h.append(hist[pl.ds(u * NB + j * 16, 16)])
                hist[pl.ds(u * NB + j * 16, 16)] = zero16
            g = h[0]
            for u in range(1, NCHUNK):
                g = g + h[u]
            e = plsc.cumsum(g)
            b = base + (e - g)
            for u in range(NCHUNK):
                fills[u][pl.ds(j * 16, 16)] = b
                b = b + h[u]
            tmp[...] = base + e
            return plsc.load_gather(tmp, [idx15])

        lax.fori_loop(0, NB // 16, body, zero16)

    def radix_pass(src, dst, p, hist_next):
        shift = jnp.int32(8 * p)
        shift2 = jnp.int32(8 * (p + 1))

        def body(i, c):
            for u in range(NCHUNK):
                off = (u * VPC + i) * 16
                k = src[pl.ds(off, 16)]
                d = lax.bitwise_and(lax.shift_right_logical(k, shift), MASK8)
                cnt, last = plsc.scan_count(d)
                cur = plsc.load_gather(fills[u], [d])
                nxt = cur + cnt
                pos = nxt - 1
                plsc.store_scatter(dst, [pos], k)
                plsc.store_scatter(fills[u], [d], nxt, mask=last)
                if hist_next is not None:
                    d2 = lax.bitwise_and(lax.shift_right_logical(k, shift2), MASK8)
                    chunk = lax.shift_right_logical(pos * CMUL, 26)
                    plsc.addupdate_scatter(
                        hist_next, [lax.bitwise_or(lax.shift_left(chunk, 8), d2)], ones16
                    )
            return c

        lax.fori_loop(0, VPC, body, 0, unroll=2)

    def sort_row(dst_final):
        conv_hist(HA)
        srcs = (bufA, bufB, bufA, bufB)
        dsts = (bufB, bufA, bufB, dst_final)
        for p in range(4):
            scan_fills(hists[p % 2])
            radix_pass(srcs[p], dsts[p], p, hists[1 - p % 2] if p < 3 else None)

    def chan_body(t, acc):
        ch = wid * CPW + t
        pltpu.sync_copy(s_hbm.at[ch], rowf)
        sort_row(sortS)
        pltpu.sync_copy(r_hbm.at[ch], rowf)
        sort_row(bufA)

        def dbody(i, a):
            fa = from_sortable(sortS[pl.ds(i * 16, 16)])
            fb = from_sortable(bufA[pl.ds(i * 16, 16)])
            d = fa - fb
            return a + d * d

        return lax.fori_loop(0, NV, dbody, acc, unroll=4)

    zero_hists()
    acc = lax.fori_loop(0, CPW, chan_body, jnp.zeros((16,), jnp.float32))
    accv[...] = acc
    pltpu.sync_copy(accv, out_hbm.at[wid])


def kernel(input_tensor, target):
    s = input_tensor.reshape(NCH, NP)
    r = target.reshape(NCH, NP)
    partials = _hist_loss_kernel(s, r)
    loss = jnp.sum(partials) / jnp.float32(N * C * H * W)
    return input_tensor, loss


# speedup vs baseline: 4.3887x; 1.0153x over previous
import functools

import jax
import jax.numpy as jnp
from jax import lax
from jax.experimental import pallas as pl
from jax.experimental.pallas import tpu as pltpu
from jax.experimental.pallas import tpu_sc as plsc

N, C, H, W = 8, 192, 56, 56
NP = H * W
NV = NP // 16
NCH = N * C
NWORK = 32
CPW = NCH // NWORK
NB = 256
NCHUNK = 7
VPC = NV // NCHUNK
CE = VPC * 16
CMUL = jnp.int32((1 << 26) // CE + 1)
MININT = jnp.int32(-(2 ** 31))
MASK8 = jnp.int32(255)

_mesh = plsc.VectorSubcoreMesh(core_axis_name="c", subcore_axis_name="s")


@functools.partial(
    pl.kernel,
    out_type=jax.ShapeDtypeStruct((NWORK, 16), jnp.float32),
    mesh=_mesh,
    scratch_types=[
        pltpu.VMEM((NP,), jnp.float32),
        pltpu.VMEM((NP,), jnp.float32),
        pltpu.VMEM((NP,), jnp.int32),
        pltpu.VMEM((NP,), jnp.int32),
        pltpu.VMEM((NP,), jnp.int32),
        pltpu.VMEM((NCHUNK * NB,), jnp.int32),
        pltpu.VMEM((NCHUNK * NB,), jnp.int32),
    ]
    + [pltpu.VMEM((NB,), jnp.int32) for _ in range(NCHUNK)]
    + [
        pltpu.VMEM((16,), jnp.int32),
        pltpu.VMEM((16,), jnp.float32),
        pltpu.SemaphoreType.DMA,
        pltpu.SemaphoreType.DMA,
    ],
    compiler_params=pltpu.CompilerParams(needs_layout_passes=False),
)
def _hist_loss_kernel(
    s_hbm, r_hbm, out_hbm,
    rowfS, rowfR, bufA, bufB, sortS, HA, HB, *rest,
):
    fills = rest[:NCHUNK]
    tmp, accv, semS, semR = rest[NCHUNK:NCHUNK + 4]
    cid = lax.axis_index("c")
    sid = lax.axis_index("s")
    wid = sid * 2 + cid
    zero16 = jnp.zeros((16,), jnp.int32)
    ones16 = jnp.ones((16,), jnp.int32)
    idx15 = jnp.full((16,), 15, jnp.int32)
    hists = (HA, HB)

    def to_sortable(b):
        return lax.bitwise_xor(b, lax.bitwise_or(lax.shift_right_arithmetic(b, 31), MININT))

    def from_sortable(k):
        b = lax.bitwise_xor(
            k, lax.bitwise_or(lax.bitwise_not(lax.shift_right_arithmetic(k, 31)), MININT)
        )
        return plsc.bitcast(b, jnp.float32)

    def dma_s(t):
        return pltpu.make_async_copy(s_hbm.at[wid * CPW + t], rowfS, semS)

    def dma_r(t):
        return pltpu.make_async_copy(r_hbm.at[wid * CPW + t], rowfR, semR)

    def zero_hists():
        def body(j, c):
            HA[pl.ds(j * 16, 16)] = zero16
            HB[pl.ds(j * 16, 16)] = zero16
            return c

        lax.fori_loop(0, NCHUNK * NB // 16, body, 0)

    def key_at(src, off, convert):
        if convert:
            return to_sortable(plsc.bitcast(src[pl.ds(off, 16)], jnp.int32))
        return src[pl.ds(off, 16)]

    def pre_hist(src, hist):
        def body(i, c):
            for u in range(NCHUNK):
                d = lax.bitwise_and(key_at(src, (u * VPC + i) * 16, True), MASK8)
                plsc.addupdate_scatter(hist, [d + jnp.int32(u * NB)], ones16)
            return c

        lax.fori_loop(0, VPC, body, 0, unroll=2)

    def scan_fills(hist):
        def body(j, base):
            h = []
            for u in range(NCHUNK):
                h.append(hist[pl.ds(u * NB + j * 16, 16)])
                hist[pl.ds(u * NB + j * 16, 16)] = zero16
            g = h[0]
            for u in range(1, NCHUNK):
                g = g + h[u]
            e = plsc.cumsum(g)
            b = base + (e - g)
            for u in range(NCHUNK):
                fills[u][pl.ds(j * 16, 16)] = b
                b = b + h[u]
            tmp[...] = base + e
            return plsc.load_gather(tmp, [idx15])

        lax.fori_loop(0, NB // 16, body, zero16)

    def radix_pass(src, dst, p, hist_next, convert=False):
        shift = jnp.int32(8 * p)
        shift2 = jnp.int32(8 * (p + 1))

        def body(i, c):
            for u in range(NCHUNK):
                k = key_at(src, (u * VPC + i) * 16, convert)
                d = lax.bitwise_and(lax.shift_right_logical(k, shift), MASK8)
                cnt, last = plsc.scan_count(d)
                cur = plsc.load_gather(fills[u], [d])
                nxt = cur + cnt
                pos = nxt - 1
                plsc.store_scatter(dst, [pos], k)
                plsc.store_scatter(fills[u], [d], nxt, mask=last)
                if hist_next is not None:
                    d2 = lax.bitwise_and(lax.shift_right_logical(k, shift2), MASK8)
                    chunk = lax.shift_right_logical(pos * CMUL, 26)
                    plsc.addupdate_scatter(
                        hist_next, [lax.bitwise_or(lax.shift_left(chunk, 8), d2)], ones16
                    )
            return c

        lax.fori_loop(0, VPC, body, 0, unroll=2)

    def sort_row(rowf, dst_final):
        pre_hist(rowf, HA)
        scan_fills(HA)
        radix_pass(rowf, bufA, 0, HB, convert=True)
        scan_fills(HB)
        radix_pass(bufA, bufB, 1, HA)
        scan_fills(HA)
        radix_pass(bufB, bufA, 2, HB)
        scan_fills(HB)
        radix_pass(bufA, dst_final, 3, None)

    def chan_body(t, acc):
        dma_s(t).wait()
        sort_row(rowfS, sortS)

        @pl.when(t + 1 < CPW)
        def _():
            dma_s(t + 1).start()

        dma_r(t).wait()
        sort_row(rowfR, bufB)

        @pl.when(t + 1 < CPW)
        def _():
            dma_r(t + 1).start()

        def dbody(i, a):
            fa = from_sortable(sortS[pl.ds(i * 16, 16)])
            fb = from_sortable(bufB[pl.ds(i * 16, 16)])
            d = fa - fb
            return a + d * d

        return lax.fori_loop(0, NV, dbody, acc, unroll=4)

    zero_hists()
    dma_s(0).start()
    dma_r(0).start()
    acc = lax.fori_loop(0, CPW, chan_body, jnp.zeros((16,), jnp.float32))
    accv[...] = acc
    pltpu.sync_copy(accv, out_hbm.at[wid])


def kernel(input_tensor, target):
    s = input_tensor.reshape(NCH, NP)
    r = target.reshape(NCH, NP)
    partials = _hist_loss_kernel(s, r)
    loss = jnp.sum(partials) / jnp.float32(N * C * H * W)
    return input_tensor, loss


# speedup vs baseline: 4.4650x; 1.0174x over previous
import functools

import jax
import jax.numpy as jnp
from jax import lax
from jax.experimental import pallas as pl
from jax.experimental.pallas import tpu as pltpu
from jax.experimental.pallas import tpu_sc as plsc

N, C, H, W = 8, 192, 56, 56
NP = H * W
NV = NP // 16
NCH = N * C
NWORK = 32
CPW = NCH // NWORK
NB = 256
NCHUNK = 4
VPC = NV // NCHUNK
CE = VPC * 16
CMUL = jnp.int32((1 << 26) // CE + 1)
MININT = jnp.int32(-(2 ** 31))
MASK8 = jnp.int32(255)

_mesh = plsc.VectorSubcoreMesh(core_axis_name="c", subcore_axis_name="s")


@functools.partial(
    pl.kernel,
    out_type=jax.ShapeDtypeStruct((NWORK, 16), jnp.float32),
    mesh=_mesh,
    scratch_types=[
        pltpu.VMEM((NP,), jnp.float32),
        pltpu.VMEM((NP,), jnp.float32),
        pltpu.VMEM((NP,), jnp.int32),
        pltpu.VMEM((NP,), jnp.int32),
        pltpu.VMEM((NP,), jnp.int32),
        pltpu.VMEM((NCHUNK * NB,), jnp.int32),
        pltpu.VMEM((NCHUNK * NB,), jnp.int32),
    ]
    + [pltpu.VMEM((NB,), jnp.int32) for _ in range(NCHUNK)]
    + [
        pltpu.VMEM((16,), jnp.int32),
        pltpu.VMEM((16,), jnp.float32),
        pltpu.SemaphoreType.DMA,
        pltpu.SemaphoreType.DMA,
    ],
    compiler_params=pltpu.CompilerParams(needs_layout_passes=False),
)
def _hist_loss_kernel(
    s_hbm, r_hbm, out_hbm,
    rowfS, rowfR, bufA, bufB, sortS, HA, HB, *rest,
):
    fills = rest[:NCHUNK]
    tmp, accv, semS, semR = rest[NCHUNK:NCHUNK + 4]
    cid = lax.axis_index("c")
    sid = lax.axis_index("s")
    wid = sid * 2 + cid
    zero16 = jnp.zeros((16,), jnp.int32)
    ones16 = jnp.ones((16,), jnp.int32)
    idx15 = jnp.full((16,), 15, jnp.int32)
    hists = (HA, HB)

    def to_sortable(b):
        return lax.bitwise_xor(b, lax.bitwise_or(lax.shift_right_arithmetic(b, 31), MININT))

    def from_sortable(k):
        b = lax.bitwise_xor(
            k, lax.bitwise_or(lax.bitwise_not(lax.shift_right_arithmetic(k, 31)), MININT)
        )
        return plsc.bitcast(b, jnp.float32)

    def dma_s(t):
        return pltpu.make_async_copy(s_hbm.at[wid * CPW + t], rowfS, semS)

    def dma_r(t):
        return pltpu.make_async_copy(r_hbm.at[wid * CPW + t], rowfR, semR)

    def zero_hists():
        def body(j, c):
            HA[pl.ds(j * 16, 16)] = zero16
            HB[pl.ds(j * 16, 16)] = zero16
            return c

        lax.fori_loop(0, NCHUNK * NB // 16, body, 0)

    def key_at(src, off, convert):
        if convert:
            return to_sortable(plsc.bitcast(src[pl.ds(off, 16)], jnp.int32))
        return src[pl.ds(off, 16)]

    def pre_hist(src, hist):
        def body(i, c):
            for u in range(NCHUNK):
                d = lax.bitwise_and(key_at(src, (u * VPC + i) * 16, True), MASK8)
                plsc.addupdate_scatter(hist, [d + jnp.int32(u * NB)], ones16)
            return c

        lax.fori_loop(0, VPC, body, 0, unroll=2)

    def scan_fills(hist):
        def body(j, base):
            h = []
            for u in range(NCHUNK):
                h.append(hist[pl.ds(u * NB + j * 16, 16)])
                hist[pl.ds(u * NB + j * 16, 16)] = zero16
            g = h[0]
            for u in range(1, NCHUNK):
                g = g + h[u]
            e = plsc.cumsum(g)
            b = base + (e - g)
            for u in range(NCHUNK):
                fills[u][pl.ds(j * 16, 16)] = b
                b = b + h[u]
            tmp[...] = base + e
            return plsc.load_gather(tmp, [idx15])

        lax.fori_loop(0, NB // 16, body, zero16)

    def radix_pass(src, dst, p, hist_next, convert=False):
        shift = jnp.int32(8 * p)
        shift2 = jnp.int32(8 * (p + 1))

        def body(i, c):
            for u in range(NCHUNK):
                k = key_at(src, (u * VPC + i) * 16, convert)
                d = lax.bitwise_and(lax.shift_right_logical(k, shift), MASK8)
                cnt, last = plsc.scan_count(d)
                cur = plsc.load_gather(fills[u], [d])
                nxt = cur + cnt
                pos = nxt - 1
                plsc.store_scatter(dst, [pos], k)
                plsc.store_scatter(fills[u], [d], nxt, mask=last)
                if hist_next is not None:
                    d2 = lax.bitwise_and(lax.shift_right_logical(k, shift2), MASK8)
                    chunk = lax.shift_right_logical(pos * CMUL, 26)
                    plsc.addupdate_scatter(
                        hist_next, [lax.bitwise_or(lax.shift_left(chunk, 8), d2)], ones16
                    )
            return c

        lax.fori_loop(0, VPC, body, 0, unroll=2)

    def sort_row(rowf, dst_final):
        pre_hist(rowf, HA)
        scan_fills(HA)
        radix_pass(rowf, bufA, 0, HB, convert=True)
        scan_fills(HB)
        radix_pass(bufA, bufB, 1, HA)
        scan_fills(HA)
        radix_pass(bufB, bufA, 2, HB)
        scan_fills(HB)
        radix_pass(bufA, dst_final, 3, None)

    def chan_body(t, acc):
        dma_s(t).wait()
        sort_row(rowfS, sortS)

        @pl.when(t + 1 < CPW)
        def _():
            dma_s(t + 1).start()

        dma_r(t).wait()
        sort_row(rowfR, bufB)

        @pl.when(t + 1 < CPW)
        def _():
            dma_r(t + 1).start()

        def dbody(i, a):
            fa = from_sortable(sortS[pl.ds(i * 16, 16)])
            fb = from_sortable(bufB[pl.ds(i * 16, 16)])
            d = fa - fb
            return a + d * d

        return lax.fori_loop(0, NV, dbody, acc, unroll=4)

    zero_hists()
    dma_s(0).start()
    dma_r(0).start()
    acc = lax.fori_loop(0, CPW, chan_body, jnp.zeros((16,), jnp.float32))
    accv[...] = acc
    pltpu.sync_copy(accv, out_hbm.at[wid])


def kernel(input_tensor, target):
    s = input_tensor.reshape(NCH, NP)
    r = target.reshape(NCH, NP)
    partials = _hist_loss_kernel(s, r)
    loss = jnp.sum(partials) / jnp.float32(N * C * H * W)
    return input_tensor, loss


# speedup vs baseline: 4.5344x; 1.0155x over previous
import functools

import jax
import jax.numpy as jnp
from jax import lax
from jax.experimental import pallas as pl
from jax.experimental.pallas import tpu as pltpu
from jax.experimental.pallas import tpu_sc as plsc

N, C, H, W = 8, 192, 56, 56
NP = H * W
NV = NP // 16
NCH = N * C
NWORK = 32
CPW = NCH // NWORK
NB = 256
NCHUNK = 2
VPC = NV // NCHUNK
CE = VPC * 16
CMUL = jnp.int32((1 << 26) // CE + 1)
MININT = jnp.int32(-(2 ** 31))
MASK8 = jnp.int32(255)

_mesh = plsc.VectorSubcoreMesh(core_axis_name="c", subcore_axis_name="s")


@functools.partial(
    pl.kernel,
    out_type=jax.ShapeDtypeStruct((NWORK, 16), jnp.float32),
    mesh=_mesh,
    scratch_types=[
        pltpu.VMEM((NP,), jnp.float32),
        pltpu.VMEM((NP,), jnp.float32),
        pltpu.VMEM((NP,), jnp.int32),
        pltpu.VMEM((NP,), jnp.int32),
        pltpu.VMEM((NP,), jnp.int32),
        pltpu.VMEM((NCHUNK * NB,), jnp.int32),
        pltpu.VMEM((NCHUNK * NB,), jnp.int32),
    ]
    + [pltpu.VMEM((NB,), jnp.int32) for _ in range(NCHUNK)]
    + [
        pltpu.VMEM((16,), jnp.int32),
        pltpu.VMEM((16,), jnp.float32),
        pltpu.SemaphoreType.DMA,
        pltpu.SemaphoreType.DMA,
    ],
    compiler_params=pltpu.CompilerParams(needs_layout_passes=False),
)
def _hist_loss_kernel(
    s_hbm, r_hbm, out_hbm,
    rowfS, rowfR, bufA, bufB, sortS, HA, HB, *rest,
):
    fills = rest[:NCHUNK]
    tmp, accv, semS, semR = rest[NCHUNK:NCHUNK + 4]
    cid = lax.axis_index("c")
    sid = lax.axis_index("s")
    wid = sid * 2 + cid
    zero16 = jnp.zeros((16,), jnp.int32)
    ones16 = jnp.ones((16,), jnp.int32)
    idx15 = jnp.full((16,), 15, jnp.int32)
    hists = (HA, HB)

    def to_sortable(b):
        return lax.bitwise_xor(b, lax.bitwise_or(lax.shift_right_arithmetic(b, 31), MININT))

    def from_sortable(k):
        b = lax.bitwise_xor(
            k, lax.bitwise_or(lax.bitwise_not(lax.shift_right_arithmetic(k, 31)), MININT)
        )
        return plsc.bitcast(b, jnp.float32)

    def dma_s(t):
        return pltpu.make_async_copy(s_hbm.at[wid * CPW + t], rowfS, semS)

    def dma_r(t):
        return pltpu.make_async_copy(r_hbm.at[wid * CPW + t], rowfR, semR)

    def zero_hists():
        def body(j, c):
            HA[pl.ds(j * 16, 16)] = zero16
            HB[pl.ds(j * 16, 16)] = zero16
            return c

        lax.fori_loop(0, NCHUNK * NB // 16, body, 0)

    def key_at(src, off, convert):
        if convert:
            return to_sortable(plsc.bitcast(src[pl.ds(off, 16)], jnp.int32))
        return src[pl.ds(off, 16)]

    def pre_hist(src, hist):
        def body(i, c):
            for u in range(NCHUNK):
                d = lax.bitwise_and(key_at(src, (u * VPC + i) * 16, True), MASK8)
                plsc.addupdate_scatter(hist, [d + jnp.int32(u * NB)], ones16)
            return c

        lax.fori_loop(0, VPC, body, 0, unroll=2)

    def scan_fills(hist):
        def body(j, base):
            h = []
            for u in range(NCHUNK):
                h.append(hist[pl.ds(u * NB + j * 16, 16)])
                hist[pl.ds(u * NB + j * 16, 16)] = zero16
            g = h[0]
            for u in range(1, NCHUNK):
                g = g + h[u]
            e = plsc.cumsum(g)
            b = base + (e - g)
            for u in range(NCHUNK):
                fills[u][pl.ds(j * 16, 16)] = b
                b = b + h[u]
            tmp[...] = base + e
            return plsc.load_gather(tmp, [idx15])

        lax.fori_loop(0, NB // 16, body, zero16)

    def radix_pass(src, dst, p, hist_next, convert=False):
        shift = jnp.int32(8 * p)
        shift2 = jnp.int32(8 * (p + 1))

        def body(i, c):
            for u in range(NCHUNK):
                k = key_at(src, (u * VPC + i) * 16, convert)
                d = lax.bitwise_and(lax.shift_right_logical(k, shift), MASK8)
                cnt, last = plsc.scan_count(d)
                cur = plsc.load_gather(fills[u], [d])
                nxt = cur + cnt
                pos = nxt - 1
                plsc.store_scatter(dst, [pos], k)
                plsc.store_scatter(fills[u], [d], nxt, mask=last)
                if hist_next is not None:
                    d2 = lax.bitwise_and(lax.shift_right_logical(k, shift2), MASK8)
                    chunk = lax.shift_right_logical(pos * CMUL, 26)
                    plsc.addupdate_scatter(
                        hist_next, [lax.bitwise_or(lax.shift_left(chunk, 8), d2)], ones16
                    )
            return c

        lax.fori_loop(0, VPC, body, 0, unroll=2)

    def sort_row(rowf, dst_final):
        pre_hist(rowf, HA)
        scan_fills(HA)
        radix_pass(rowf, bufA, 0, HB, convert=True)
        scan_fills(HB)
        radix_pass(bufA, bufB, 1, HA)
        scan_fills(HA)
        radix_pass(bufB, bufA, 2, HB)
        scan_fills(HB)
        radix_pass(bufA, dst_final, 3, None)

    def chan_body(t, acc):
        dma_s(t).wait()
        sort_row(rowfS, sortS)

        @pl.when(t + 1 < CPW)
        def _():
            dma_s(t + 1).start()

        dma_r(t).wait()
        sort_row(rowfR, bufB)

        @pl.when(t + 1 < CPW)
        def _():
            dma_r(t + 1).start()

        def dbody(i, a):
            fa = from_sortable(sortS[pl.ds(i * 16, 16)])
            fb = from_sortable(bufB[pl.ds(i * 16, 16)])
            d = fa - fb
            return a + d * d

        return lax.fori_loop(0, NV, dbody, acc, unroll=4)

    zero_hists()
    dma_s(0).start()
    dma_r(0).start()
    acc = lax.fori_loop(0, CPW, chan_body, jnp.zeros((16,), jnp.float32))
    accv[...] = acc
    pltpu.sync_copy(accv, out_hbm.at[wid])


def kernel(input_tensor, target):
    s = input_tensor.reshape(NCH, NP)
    r = target.reshape(NCH, NP)
    partials = _hist_loss_kernel(s, r)
    loss = jnp.sum(partials) / jnp.float32(N * C * H * W)
    return input_tensor, loss


# speedup vs baseline: 4.8908x; 1.0786x over previous
import functools

import jax
import jax.numpy as jnp
from jax import lax
from jax.experimental import pallas as pl
from jax.experimental.pallas import tpu as pltpu
from jax.experimental.pallas import tpu_sc as plsc

N, C, H, W = 8, 192, 56, 56
NP = H * W
NV = NP // 16
NCH = N * C
NWORK = 32
CPW = NCH // NWORK
NB = 256
NCHUNK = 1
VPC = NV // NCHUNK
CE = VPC * 16
CMUL = jnp.int32((1 << 26) // CE + 1)
MININT = jnp.int32(-(2 ** 31))
MASK8 = jnp.int32(255)

_mesh = plsc.VectorSubcoreMesh(core_axis_name="c", subcore_axis_name="s")


@functools.partial(
    pl.kernel,
    out_type=jax.ShapeDtypeStruct((NWORK, 16), jnp.float32),
    mesh=_mesh,
    scratch_types=[
        pltpu.VMEM((NP,), jnp.float32),
        pltpu.VMEM((NP,), jnp.float32),
        pltpu.VMEM((NP,), jnp.int32),
        pltpu.VMEM((NP,), jnp.int32),
        pltpu.VMEM((NP,), jnp.int32),
        pltpu.VMEM((NCHUNK * NB,), jnp.int32),
        pltpu.VMEM((NCHUNK * NB,), jnp.int32),
    ]
    + [pltpu.VMEM((NB,), jnp.int32) for _ in range(NCHUNK)]
    + [
        pltpu.VMEM((16,), jnp.int32),
        pltpu.VMEM((16,), jnp.float32),
        pltpu.SemaphoreType.DMA,
        pltpu.SemaphoreType.DMA,
    ],
    compiler_params=pltpu.CompilerParams(needs_layout_passes=False),
)
def _hist_loss_kernel(
    s_hbm, r_hbm, out_hbm,
    rowfS, rowfR, bufA, bufB, sortS, HA, HB, *rest,
):
    fills = rest[:NCHUNK]
    tmp, accv, semS, semR = rest[NCHUNK:NCHUNK + 4]
    cid = lax.axis_index("c")
    sid = lax.axis_index("s")
    wid = sid * 2 + cid
    zero16 = jnp.zeros((16,), jnp.int32)
    ones16 = jnp.ones((16,), jnp.int32)
    idx15 = jnp.full((16,), 15, jnp.int32)
    hists = (HA, HB)

    def to_sortable(b):
        return lax.bitwise_xor(b, lax.bitwise_or(lax.shift_right_arithmetic(b, 31), MININT))

    def from_sortable(k):
        b = lax.bitwise_xor(
            k, lax.bitwise_or(lax.bitwise_not(lax.shift_right_arithmetic(k, 31)), MININT)
        )
        return plsc.bitcast(b, jnp.float32)

    def dma_s(t):
        return pltpu.make_async_copy(s_hbm.at[wid * CPW + t], rowfS, semS)

    def dma_r(t):
        return pltpu.make_async_copy(r_hbm.at[wid * CPW + t], rowfR, semR)

    def zero_hists():
        def body(j, c):
            HA[pl.ds(j * 16, 16)] = zero16
            HB[pl.ds(j * 16, 16)] = zero16
            return c

        lax.fori_loop(0, NCHUNK * NB // 16, body, 0)

    def key_at(src, off, convert):
        if convert:
            return to_sortable(plsc.bitcast(src[pl.ds(off, 16)], jnp.int32))
        return src[pl.ds(off, 16)]

    def pre_hist(src, hist):
        def body(i, c):
            for u in range(NCHUNK):
                d = lax.bitwise_and(key_at(src, (u * VPC + i) * 16, True), MASK8)
                plsc.addupdate_scatter(hist, [d + jnp.int32(u * NB)], ones16)
            return c

        lax.fori_loop(0, VPC, body, 0, unroll=2)

    def scan_fills(hist):
        def body(j, base):
            h = []
            for u in range(NCHUNK):
                h.append(hist[pl.ds(u * NB + j * 16, 16)])
                hist[pl.ds(u * NB + j * 16, 16)] = zero16
            g = h[0]
            for u in range(1, NCHUNK):
                g = g + h[u]
            e = plsc.cumsum(g)
            b = base + (e - g)
            for u in range(NCHUNK):
                fills[u][pl.ds(j * 16, 16)] = b
                b = b + h[u]
            tmp[...] = base + e
            return plsc.load_gather(tmp, [idx15])

        lax.fori_loop(0, NB // 16, body, zero16)

    def radix_pass(src, dst, p, hist_next, convert=False):
        shift = jnp.int32(8 * p)
        shift2 = jnp.int32(8 * (p + 1))

        def body(i, c):
            for u in range(NCHUNK):
                k = key_at(src, (u * VPC + i) * 16, convert)
                d = lax.bitwise_and(lax.shift_right_logical(k, shift), MASK8)
                cnt, last = plsc.scan_count(d)
                cur = plsc.load_gather(fills[u], [d])
                nxt = cur + cnt
                pos = nxt - 1
                plsc.store_scatter(dst, [pos], k)
                plsc.store_scatter(fills[u], [d], nxt, mask=last)
                if hist_next is not None:
                    d2 = lax.bitwise_and(lax.shift_right_logical(k, shift2), MASK8)
                    if NCHUNK == 1:
                        plsc.addupdate_scatter(hist_next, [d2], ones16)
                    else:
                        chunk = lax.shift_right_logical(pos * CMUL, 26)
                        plsc.addupdate_scatter(
                            hist_next, [lax.bitwise_or(lax.shift_left(chunk, 8), d2)], ones16
                        )
            return c

        lax.fori_loop(0, VPC, body, 0, unroll=2)

    def sort_row(rowf, dst_final):
        pre_hist(rowf, HA)
        scan_fills(HA)
        radix_pass(rowf, bufA, 0, HB, convert=True)
        scan_fills(HB)
        radix_pass(bufA, bufB, 1, HA)
        scan_fills(HA)
        radix_pass(bufB, bufA, 2, HB)
        scan_fills(HB)
        radix_pass(bufA, dst_final, 3, None)

    def chan_body(t, acc):
        dma_s(t).wait()
        sort_row(rowfS, sortS)

        @pl.when(t + 1 < CPW)
        def _():
            dma_s(t + 1).start()

        dma_r(t).wait()
        sort_row(rowfR, bufB)

        @pl.when(t + 1 < CPW)
        def _():
            dma_r(t + 1).start()

        def dbody(i, a):
            fa = from_sortable(sortS[pl.ds(i * 16, 16)])
            fb = from_sortable(bufB[pl.ds(i * 16, 16)])
            d = fa - fb
            return a + d * d

        return lax.fori_loop(0, NV, dbody, acc, unroll=4)

    zero_hists()
    dma_s(0).start()
    dma_r(0).start()
    acc = lax.fori_loop(0, CPW, chan_body, jnp.zeros((16,), jnp.float32))
    accv[...] = acc
    pltpu.sync_copy(accv, out_hbm.at[wid])


def kernel(input_tensor, target):
    s = input_tensor.reshape(NCH, NP)
    r = target.reshape(NCH, NP)
    partials = _hist_loss_kernel(s, r)
    loss = jnp.sum(partials) / jnp.float32(N * C * H * W)
    return input_tensor, loss


# speedup vs baseline: 4.8978x; 1.0014x over previous
import functools

import jax
import jax.numpy as jnp
from jax import lax
from jax.experimental import pallas as pl
from jax.experimental.pallas import tpu as pltpu
from jax.experimental.pallas import tpu_sc as plsc

N, C, H, W = 8, 192, 56, 56
NP = H * W
NV = NP // 16
NCH = N * C
NWORK = 32
CPW = NCH // NWORK
NB = 256
NCHUNK = 1
VPC = NV // NCHUNK
CE = VPC * 16
CMUL = jnp.int32((1 << 26) // CE + 1)
MININT = jnp.int32(-(2 ** 31))
MASK8 = jnp.int32(255)

_mesh = plsc.VectorSubcoreMesh(core_axis_name="c", subcore_axis_name="s")


@functools.partial(
    pl.kernel,
    out_type=jax.ShapeDtypeStruct((NWORK, 16), jnp.float32),
    mesh=_mesh,
    scratch_types=[
        pltpu.VMEM((NP,), jnp.float32),
        pltpu.VMEM((NP,), jnp.float32),
        pltpu.VMEM((NP,), jnp.int32),
        pltpu.VMEM((NP,), jnp.int32),
        pltpu.VMEM((NP,), jnp.int32),
        pltpu.VMEM((NCHUNK * NB,), jnp.int32),
        pltpu.VMEM((NCHUNK * NB,), jnp.int32),
    ]
    + [pltpu.VMEM((NB,), jnp.int32) for _ in range(NCHUNK)]
    + [
        pltpu.VMEM((16,), jnp.int32),
        pltpu.VMEM((16,), jnp.float32),
        pltpu.SemaphoreType.DMA,
        pltpu.SemaphoreType.DMA,
    ],
    compiler_params=pltpu.CompilerParams(needs_layout_passes=False),
)
def _hist_loss_kernel(
    s_hbm, r_hbm, out_hbm,
    rowfS, rowfR, bufA, bufB, sortS, HA, HB, *rest,
):
    fills = rest[:NCHUNK]
    tmp, accv, semS, semR = rest[NCHUNK:NCHUNK + 4]
    cid = lax.axis_index("c")
    sid = lax.axis_index("s")
    wid = sid * 2 + cid
    zero16 = jnp.zeros((16,), jnp.int32)
    ones16 = jnp.ones((16,), jnp.int32)
    idx15 = jnp.full((16,), 15, jnp.int32)
    hists = (HA, HB)

    def to_sortable(b):
        return lax.bitwise_xor(b, lax.bitwise_or(lax.shift_right_arithmetic(b, 31), MININT))

    def from_sortable(k):
        b = lax.bitwise_xor(
            k, lax.bitwise_or(lax.bitwise_not(lax.shift_right_arithmetic(k, 31)), MININT)
        )
        return plsc.bitcast(b, jnp.float32)

    def dma_s(t):
        return pltpu.make_async_copy(s_hbm.at[wid * CPW + t], rowfS, semS)

    def dma_r(t):
        return pltpu.make_async_copy(r_hbm.at[wid * CPW + t], rowfR, semR)

    def zero_hists():
        def body(j, c):
            HA[pl.ds(j * 16, 16)] = zero16
            HB[pl.ds(j * 16, 16)] = zero16
            return c

        lax.fori_loop(0, NCHUNK * NB // 16, body, 0)

    def key_at(src, off, convert):
        if convert:
            return to_sortable(plsc.bitcast(src[pl.ds(off, 16)], jnp.int32))
        return src[pl.ds(off, 16)]

    def pre_hist(src, hist):
        def body(i, c):
            for u in range(NCHUNK):
                d = lax.bitwise_and(key_at(src, (u * VPC + i) * 16, True), MASK8)
                plsc.addupdate_scatter(hist, [d + jnp.int32(u * NB)], ones16)
            return c

        lax.fori_loop(0, VPC, body, 0, unroll=2)

    def scan_fills(hist):
        def body(j, base):
            h = []
            for u in range(NCHUNK):
                h.append(hist[pl.ds(u * NB + j * 16, 16)])
                hist[pl.ds(u * NB + j * 16, 16)] = zero16
            g = h[0]
            for u in range(1, NCHUNK):
                g = g + h[u]
            e = plsc.cumsum(g)
            b = base + (e - g)
            for u in range(NCHUNK):
                fills[u][pl.ds(j * 16, 16)] = b
                b = b + h[u]
            tmp[...] = base + e
            return plsc.load_gather(tmp, [idx15])

        lax.fori_loop(0, NB // 16, body, zero16)

    def radix_pass(src, dst, p, hist_next, convert=False):
        shift = jnp.int32(8 * p)
        shift2 = jnp.int32(8 * (p + 1))

        def body(i, c):
            for u in range(NCHUNK):
                k = key_at(src, (u * VPC + i) * 16, convert)
                d = lax.bitwise_and(lax.shift_right_logical(k, shift), MASK8)
                cnt, last = plsc.scan_count(d)
                cur = plsc.load_gather(fills[u], [d])
                nxt = cur + cnt
                pos = nxt - 1
                plsc.store_scatter(dst, [pos], k)
                plsc.store_scatter(fills[u], [d], nxt, mask=last)
                if hist_next is not None:
                    d2 = lax.bitwise_and(lax.shift_right_logical(k, shift2), MASK8)
                    if NCHUNK == 1:
                        plsc.addupdate_scatter(hist_next, [d2], ones16)
                    else:
                        chunk = lax.shift_right_logical(pos * CMUL, 26)
                        plsc.addupdate_scatter(
                            hist_next, [lax.bitwise_or(lax.shift_left(chunk, 8), d2)], ones16
                        )
            return c

        lax.fori_loop(0, VPC, body, 0, unroll=4)

    def sort_row(rowf, dst_final):
        pre_hist(rowf, HA)
        scan_fills(HA)
        radix_pass(rowf, bufA, 0, HB, convert=True)
        scan_fills(HB)
        radix_pass(bufA, bufB, 1, HA)
        scan_fills(HA)
        radix_pass(bufB, bufA, 2, HB)
        scan_fills(HB)
        radix_pass(bufA, dst_final, 3, None)

    def chan_body(t, acc):
        dma_s(t).wait()
        sort_row(rowfS, sortS)

        @pl.when(t + 1 < CPW)
        def _():
            dma_s(t + 1).start()

        dma_r(t).wait()
        sort_row(rowfR, bufB)

        @pl.when(t + 1 < CPW)
        def _():
            dma_r(t + 1).start()

        def dbody(i, a):
            fa = from_sortable(sortS[pl.ds(i * 16, 16)])
            fb = from_sortable(bufB[pl.ds(i * 16, 16)])
            d = fa - fb
            return a + d * d

        return lax.fori_loop(0, NV, dbody, acc, unroll=4)

    zero_hists()
    dma_s(0).start()
    dma_r(0).start()
    acc = lax.fori_loop(0, CPW, chan_body, jnp.zeros((16,), jnp.float32))
    accv[...] = acc
    pltpu.sync_copy(accv, out_hbm.at[wid])


def kernel(input_tensor, target):
    s = input_tensor.reshape(NCH, NP)
    r = target.reshape(NCH, NP)
    partials = _hist_loss_kernel(s, r)
    loss = jnp.sum(partials) / jnp.float32(N * C * H * W)
    return input_tensor, loss


# speedup vs baseline: 4.9203x; 1.0046x over previous
import functools

import jax
import jax.numpy as jnp
from jax import lax
from jax.experimental import pallas as pl
from jax.experimental.pallas import tpu as pltpu
from jax.experimental.pallas import tpu_sc as plsc

N, C, H, W = 8, 192, 56, 56
NP = H * W
NV = NP // 16
NCH = N * C
NWORK = 32
CPW = NCH // NWORK
NB = 256
NCHUNK = 1
VPC = NV // NCHUNK
CE = VPC * 16
CMUL = jnp.int32((1 << 26) // CE + 1)
MININT = jnp.int32(-(2 ** 31))
MASK8 = jnp.int32(255)

_mesh = plsc.VectorSubcoreMesh(core_axis_name="c", subcore_axis_name="s")


@functools.partial(
    pl.kernel,
    out_type=jax.ShapeDtypeStruct((NWORK, 16), jnp.float32),
    mesh=_mesh,
    scratch_types=[
        pltpu.VMEM((NP,), jnp.float32),
        pltpu.VMEM((NP,), jnp.float32),
        pltpu.VMEM((NP,), jnp.int32),
        pltpu.VMEM((NP,), jnp.int32),
        pltpu.VMEM((NP,), jnp.float32),
        pltpu.VMEM((NCHUNK * NB,), jnp.int32),
        pltpu.VMEM((NCHUNK * NB,), jnp.int32),
    ]
    + [pltpu.VMEM((NB,), jnp.int32) for _ in range(NCHUNK)]
    + [
        pltpu.VMEM((16,), jnp.int32),
        pltpu.VMEM((16,), jnp.float32),
        pltpu.SemaphoreType.DMA,
        pltpu.SemaphoreType.DMA,
    ],
    compiler_params=pltpu.CompilerParams(needs_layout_passes=False),
)
def _hist_loss_kernel(
    s_hbm, r_hbm, out_hbm,
    rowfS, rowfR, bufA, bufB, sortS, HA, HB, *rest,
):
    fills = rest[:NCHUNK]
    tmp, accv, semS, semR = rest[NCHUNK:NCHUNK + 4]
    cid = lax.axis_index("c")
    sid = lax.axis_index("s")
    wid = sid * 2 + cid
    zero16 = jnp.zeros((16,), jnp.int32)
    ones16 = jnp.ones((16,), jnp.int32)
    idx15 = jnp.full((16,), 15, jnp.int32)
    hists = (HA, HB)

    def to_sortable(b):
        return lax.bitwise_xor(b, lax.bitwise_or(lax.shift_right_arithmetic(b, 31), MININT))

    def from_sortable(k):
        b = lax.bitwise_xor(
            k, lax.bitwise_or(lax.bitwise_not(lax.shift_right_arithmetic(k, 31)), MININT)
        )
        return plsc.bitcast(b, jnp.float32)

    def dma_s(t):
        return pltpu.make_async_copy(s_hbm.at[wid * CPW + t], rowfS, semS)

    def dma_r(t):
        return pltpu.make_async_copy(r_hbm.at[wid * CPW + t], rowfR, semR)

    def zero_hists():
        def body(j, c):
            HA[pl.ds(j * 16, 16)] = zero16
            HB[pl.ds(j * 16, 16)] = zero16
            return c

        lax.fori_loop(0, NCHUNK * NB // 16, body, 0)

    def key_at(src, off, convert):
        if convert:
            return to_sortable(plsc.bitcast(src[pl.ds(off, 16)], jnp.int32))
        return src[pl.ds(off, 16)]

    def pre_hist(src, hist):
        def body(i, c):
            for u in range(NCHUNK):
                d = lax.bitwise_and(key_at(src, (u * VPC + i) * 16, True), MASK8)
                plsc.addupdate_scatter(hist, [d + jnp.int32(u * NB)], ones16)
            return c

        lax.fori_loop(0, VPC, body, 0, unroll=2)

    def scan_fills(hist):
        def body(j, base):
            h = []
            for u in range(NCHUNK):
                h.append(hist[pl.ds(u * NB + j * 16, 16)])
                hist[pl.ds(u * NB + j * 16, 16)] = zero16
            g = h[0]
            for u in range(1, NCHUNK):
                g = g + h[u]
            e = plsc.cumsum(g)
            b = base + (e - g)
            for u in range(NCHUNK):
                fills[u][pl.ds(j * 16, 16)] = b
                b = b + h[u]
            tmp[...] = base + e
            return plsc.load_gather(tmp, [idx15])

        lax.fori_loop(0, NB // 16, body, zero16)

    def radix_pass(src, dst, p, hist_next, convert=False, mode="keys", acc=None):
        shift = jnp.int32(8 * p)
        shift2 = jnp.int32(8 * (p + 1))

        def body(i, c):
            for u in range(NCHUNK):
                k = key_at(src, (u * VPC + i) * 16, convert)
                d = lax.bitwise_and(lax.shift_right_logical(k, shift), MASK8)
                cnt, last = plsc.scan_count(d)
                cur = plsc.load_gather(fills[u], [d])
                nxt = cur + cnt
                pos = nxt - 1
                if mode == "keys":
                    plsc.store_scatter(dst, [pos], k)
                elif mode == "f32":
                    plsc.store_scatter(dst, [pos], from_sortable(k))
                else:
                    fa = plsc.load_gather(sortS, [pos])
                    dd = fa - from_sortable(k)
                    c = c + dd * dd
                plsc.store_scatter(fills[u], [d], nxt, mask=last)
                if hist_next is not None:
                    d2 = lax.bitwise_and(lax.shift_right_logical(k, shift2), MASK8)
                    if NCHUNK == 1:
                        plsc.addupdate_scatter(hist_next, [d2], ones16)
                    else:
                        chunk = lax.shift_right_logical(pos * CMUL, 26)
                        plsc.addupdate_scatter(
                            hist_next, [lax.bitwise_or(lax.shift_left(chunk, 8), d2)], ones16
                        )
            return c

        init = acc if mode == "diff" else 0
        return lax.fori_loop(0, VPC, body, init, unroll=4)

    def sort_row(rowf, mode, acc=None):
        pre_hist(rowf, HA)
        scan_fills(HA)
        radix_pass(rowf, bufA, 0, HB, convert=True)
        scan_fills(HB)
        radix_pass(bufA, bufB, 1, HA)
        scan_fills(HA)
        radix_pass(bufB, bufA, 2, HB)
        scan_fills(HB)
        return radix_pass(bufA, sortS, 3, None, mode=mode, acc=acc)

    def chan_body(t, acc):
        dma_s(t).wait()
        sort_row(rowfS, "f32")

        @pl.when(t + 1 < CPW)
        def _():
            dma_s(t + 1).start()

        dma_r(t).wait()
        acc = sort_row(rowfR, "diff", acc)

        @pl.when(t + 1 < CPW)
        def _():
            dma_r(t + 1).start()

        return acc

    zero_hists()
    dma_s(0).start()
    dma_r(0).start()
    acc = lax.fori_loop(0, CPW, chan_body, jnp.zeros((16,), jnp.float32))
    accv[...] = acc
    pltpu.sync_copy(accv, out_hbm.at[wid])


def kernel(input_tensor, target):
    s = input_tensor.reshape(NCH, NP)
    r = target.reshape(NCH, NP)
    partials = _hist_loss_kernel(s, r)
    loss = jnp.sum(partials) / jnp.float32(N * C * H * W)
    return input_tensor, loss


# speedup vs baseline: 4.9772x; 1.0116x over previous
import functools

import jax
import jax.numpy as jnp
from jax import lax
from jax.experimental import pallas as pl
from jax.experimental.pallas import tpu as pltpu
from jax.experimental.pallas import tpu_sc as plsc

N, C, H, W = 8, 192, 56, 56
NP = H * W
NV = NP // 16
NCH = N * C
NWORK = 32
CPW = NCH // NWORK
NB = 256
NCHUNK = 1
VPC = NV // NCHUNK
CE = VPC * 16
CMUL = jnp.int32((1 << 26) // CE + 1)
MININT = jnp.int32(-(2 ** 31))
MASK8 = jnp.int32(255)

_mesh = plsc.VectorSubcoreMesh(core_axis_name="c", subcore_axis_name="s")


@functools.partial(
    pl.kernel,
    out_type=jax.ShapeDtypeStruct((NWORK, 16), jnp.float32),
    mesh=_mesh,
    scratch_types=[
        pltpu.VMEM((NP,), jnp.float32),
        pltpu.VMEM((NP,), jnp.float32),
        pltpu.VMEM((NP,), jnp.int32),
        pltpu.VMEM((NP,), jnp.int32),
        pltpu.VMEM((NP,), jnp.float32),
        pltpu.VMEM((NCHUNK * NB,), jnp.int32),
        pltpu.VMEM((NCHUNK * NB,), jnp.int32),
    ]
    + [pltpu.VMEM((NB,), jnp.int32) for _ in range(NCHUNK)]
    + [
        pltpu.VMEM((16,), jnp.int32),
        pltpu.VMEM((16,), jnp.float32),
        pltpu.SemaphoreType.DMA,
        pltpu.SemaphoreType.DMA,
    ],
    compiler_params=pltpu.CompilerParams(needs_layout_passes=False),
)
def _hist_loss_kernel(
    s_hbm, r_hbm, out_hbm,
    rowfS, rowfR, bufA, bufB, sortS, HA, HB, *rest,
):
    fills = rest[:NCHUNK]
    tmp, accv, semS, semR = rest[NCHUNK:NCHUNK + 4]
    cid = lax.axis_index("c")
    sid = lax.axis_index("s")
    wid = sid * 2 + cid
    zero16 = jnp.zeros((16,), jnp.int32)
    ones16 = jnp.ones((16,), jnp.int32)
    idx15 = jnp.full((16,), 15, jnp.int32)
    hists = (HA, HB)

    def to_sortable(b):
        return lax.bitwise_xor(b, lax.bitwise_or(lax.shift_right_arithmetic(b, 31), MININT))

    def from_sortable(k):
        b = lax.bitwise_xor(
            k, lax.bitwise_or(lax.bitwise_not(lax.shift_right_arithmetic(k, 31)), MININT)
        )
        return plsc.bitcast(b, jnp.float32)

    def dma_s(t):
        return pltpu.make_async_copy(s_hbm.at[wid * CPW + t], rowfS, semS)

    def dma_r(t):
        return pltpu.make_async_copy(r_hbm.at[wid * CPW + t], rowfR, semR)

    def zero_hists():
        def body(j, c):
            HA[pl.ds(j * 16, 16)] = zero16
            HB[pl.ds(j * 16, 16)] = zero16
            return c

        lax.fori_loop(0, NCHUNK * NB // 16, body, 0)

    def key_at(src, off, convert):
        if convert:
            return to_sortable(plsc.bitcast(src[pl.ds(off, 16)], jnp.int32))
        return src[pl.ds(off, 16)]

    def pre_hist(src, hist):
        def body(i, c):
            for u in range(NCHUNK):
                d = lax.bitwise_and(key_at(src, (u * VPC + i) * 16, True), MASK8)
                plsc.addupdate_scatter(hist, [d + jnp.int32(u * NB)], ones16)
            return c

        lax.fori_loop(0, VPC, body, 0, unroll=2)

    def scan_fills(hist):
        def body(j, base):
            h = []
            for u in range(NCHUNK):
                h.append(hist[pl.ds(u * NB + j * 16, 16)])
                hist[pl.ds(u * NB + j * 16, 16)] = zero16
            g = h[0]
            for u in range(1, NCHUNK):
                g = g + h[u]
            e = plsc.cumsum(g)
            b = base + (e - g)
            for u in range(NCHUNK):
                fills[u][pl.ds(j * 16, 16)] = b
                b = b + h[u]
            tmp[...] = base + e
            return plsc.load_gather(tmp, [idx15])

        lax.fori_loop(0, NB // 16, body, zero16)

    def radix_pass(src, dst, p, hist_next, convert=False, mode="keys", acc=None,
                   next_rowf=None):
        shift = jnp.int32(8 * p)
        shift2 = jnp.int32(8 * (p + 1))

        def body(i, c):
            for u in range(NCHUNK):
                off = (u * VPC + i) * 16
                k = key_at(src, off, convert)
                d = lax.bitwise_and(lax.shift_right_logical(k, shift), MASK8)
                cnt, last = plsc.scan_count(d)
                cur = plsc.load_gather(fills[u], [d])
                nxt = cur + cnt
                pos = nxt - 1
                if mode == "keys":
                    plsc.store_scatter(dst, [pos], k)
                elif mode == "f32":
                    plsc.store_scatter(dst, [pos], from_sortable(k))
                else:
                    fa = plsc.load_gather(sortS, [pos])
                    dd = fa - from_sortable(k)
                    c = c + dd * dd
                plsc.store_scatter(fills[u], [d], nxt, mask=last)
                if hist_next is not None:
                    d2 = lax.bitwise_and(lax.shift_right_logical(k, shift2), MASK8)
                    if NCHUNK == 1:
                        plsc.addupdate_scatter(hist_next, [d2], ones16)
                    else:
                        chunk = lax.shift_right_logical(pos * CMUL, 26)
                        plsc.addupdate_scatter(
                            hist_next, [lax.bitwise_or(lax.shift_left(chunk, 8), d2)], ones16
                        )
                if next_rowf is not None:
                    dn = lax.bitwise_and(key_at(next_rowf, off, True), MASK8)
                    plsc.addupdate_scatter(HA, [dn], ones16)
            return c

        init = acc if mode == "diff" else 0
        return lax.fori_loop(0, VPC, body, init, unroll=4)

    def sort_row(rowf, mode, acc=None, next_rowf=None, pre_p3=None):
        scan_fills(HA)
        radix_pass(rowf, bufA, 0, HB, convert=True)
        scan_fills(HB)
        radix_pass(bufA, bufB, 1, HA)
        scan_fills(HA)
        radix_pass(bufB, bufA, 2, HB)
        scan_fills(HB)
        if pre_p3 is not None:
            pre_p3()
        return radix_pass(bufA, sortS, 3, None, mode=mode, acc=acc,
                          next_rowf=next_rowf)

    def chan_body(t, acc):
        sort_row(rowfS, "f32", next_rowf=rowfR, pre_p3=lambda: dma_r(t).wait())

        @pl.when(t + 1 < CPW)
        def _():
            dma_s(t + 1).start()

        def wait_next_s():
            @pl.when(t + 1 < CPW)
            def _():
                dma_s(t + 1).wait()

        acc = sort_row(rowfR, "diff", acc, next_rowf=rowfS, pre_p3=wait_next_s)

        @pl.when(t + 1 < CPW)
        def _():
            dma_r(t + 1).start()

        return acc

    zero_hists()
    dma_s(0).start()
    dma_r(0).start()
    dma_s(0).wait()
    pre_hist(rowfS, HA)
    acc = lax.fori_loop(0, CPW, chan_body, jnp.zeros((16,), jnp.float32))
    accv[...] = acc
    pltpu.sync_copy(accv, out_hbm.at[wid])


def kernel(input_tensor, target):
    s = input_tensor.reshape(NCH, NP)
    r = target.reshape(NCH, NP)
    partials = _hist_loss_kernel(s, r)
    loss = jnp.sum(partials) / jnp.float32(N * C * H * W)
    return input_tensor, loss
